# Initial kernel scaffold; baseline (speedup 1.0000x reference)
import jax
import jax.numpy as jnp
from jax.experimental import pallas as pl


def kernel(features, xyz_0, xyz_1, xyz_2, xyz_3, neigh_idx_0, neigh_idx_1, neigh_idx_2, neigh_idx_3, sub_idx_0, sub_idx_1, sub_idx_2, sub_idx_3, interp_idx_0, interp_idx_1, interp_idx_2, interp_idx_3, params):
    raise NotImplementedError("write your pallas kernel here")


# speedup vs baseline: 3.1134x; 3.1134x over previous
import functools
import math

import jax
import jax.numpy as jnp
from jax import lax
from jax.experimental import pallas as pl
from jax.experimental.pallas import tpu as pltpu
from jax.experimental.pallas import tpu_sc as plsc

_KNN = 16
_NS = [40960, 10240, 2560, 640, 160]
_DOUT = [16, 64, 128, 256]
_BATCH = 2


def _gather_rows(table, idx):
    return jnp.take(table, idx, axis=0)


def _blk(rows, cap):
    return math.gcd(rows, cap)


def _dense_act(x, w, b, act, slope):
    r, din = x.shape
    dout = w.shape[1]
    nb = _blk(r, 512)

    def body(x_ref, w_ref, b_ref, o_ref):
        y = jnp.dot(x_ref[...], w_ref[...],
                    preferred_element_type=jnp.float32) + b_ref[...]
        if act:
            y = jnp.where(y >= 0, y, slope * y)
        o_ref[...] = y

    return pl.pallas_call(
        body,
        grid=(r // nb,),
        in_specs=[
            pl.BlockSpec((nb, din), lambda i: (i, 0)),
            pl.BlockSpec((din, dout), lambda i: (0, 0)),
            pl.BlockSpec((1, dout), lambda i: (0, 0)),
        ],
        out_specs=pl.BlockSpec((nb, dout), lambda i: (i, 0)),
        out_shape=jax.ShapeDtypeStruct((r, dout), jnp.float32),
    )(x, w, b.reshape(1, dout))


def _dense2_act(x1, x2, w1, w2, b, act, slope):
    r, d1 = x1.shape
    d2 = x2.shape[1]
    dout = w1.shape[1]
    nb = _blk(r, 512)

    def body(x1_ref, x2_ref, w1_ref, w2_ref, b_ref, o_ref):
        y = (jnp.dot(x1_ref[...], w1_ref[...], preferred_element_type=jnp.float32)
             + jnp.dot(x2_ref[...], w2_ref[...], preferred_element_type=jnp.float32)
             + b_ref[...])
        if act:
            y = jnp.where(y >= 0, y, slope * y)
        o_ref[...] = y

    return pl.pallas_call(
        body,
        grid=(r // nb,),
        in_specs=[
            pl.BlockSpec((nb, d1), lambda i: (i, 0)),
            pl.BlockSpec((nb, d2), lambda i: (i, 0)),
            pl.BlockSpec((d1, dout), lambda i: (0, 0)),
            pl.BlockSpec((d2, dout), lambda i: (0, 0)),
            pl.BlockSpec((1, dout), lambda i: (0, 0)),
        ],
        out_specs=pl.BlockSpec((nb, dout), lambda i: (i, 0)),
        out_shape=jax.ShapeDtypeStruct((r, dout), jnp.float32),
    )(x1, x2, w1, w2, b.reshape(1, dout))


def _relpos_mlp(xyz, nbr_xyz, w, b):
    r = xyz.shape[0]
    dh = w.shape[1]
    k = _KNN
    nb = _blk(r, 256)

    def body(xyz_ref, nbr_ref, w_ref, b_ref, o_ref):
        tile = xyz_ref[...]
        tile = jnp.broadcast_to(tile[:, None, :], (nb, k, 3)).reshape(nb * k, 3)
        nbr = nbr_ref[...]
        rel = tile - nbr
        dis = jnp.sqrt(jnp.sum(rel * rel, axis=-1, keepdims=True) + 1e-12)
        feat = jnp.concatenate([dis, rel, tile, nbr], axis=-1)
        y = jnp.dot(feat, w_ref[...], preferred_element_type=jnp.float32) + b_ref[...]
        o_ref[...] = jnp.where(y >= 0, y, 0.2 * y)

    return pl.pallas_call(
        body,
        grid=(r // nb,),
        in_specs=[
            pl.BlockSpec((nb, 3), lambda i: (i, 0)),
            pl.BlockSpec((nb * k, 3), lambda i: (i, 0)),
            pl.BlockSpec((10, dh), lambda i: (0, 0)),
            pl.BlockSpec((1, dh), lambda i: (0, 0)),
        ],
        out_specs=pl.BlockSpec((nb * k, dh), lambda i: (i, 0)),
        out_shape=jax.ShapeDtypeStruct((r * k, dh), jnp.float32),
    )(xyz, nbr_xyz, w, b.reshape(1, dh))


def _att_pool(fn, fx, a1, b1, a2, b2, bf1, bf2, wm1, wm2, bm):
    rk, dh = fn.shape
    k = _KNN
    r = rk // k
    dmlp = wm1.shape[1]
    nb = _blk(r, 256)

    def body(fn_ref, fx_ref, a1_ref, b1_ref, a2_ref, b2_ref,
             bf1_ref, bf2_ref, wm1_ref, wm2_ref, bm_ref, o_ref):
        fnv = fn_ref[...]
        fxv = fx_ref[...]
        att1 = (jnp.dot(fnv, a1_ref[...], preferred_element_type=jnp.float32)
                + jnp.dot(fxv, b1_ref[...], preferred_element_type=jnp.float32)
                + bf1_ref[...])
        att2 = (jnp.dot(fnv, a2_ref[...], preferred_element_type=jnp.float32)
                + jnp.dot(fxv, b2_ref[...], preferred_element_type=jnp.float32)
                + bf2_ref[...])

        def soft_agg(att, f):
            a3 = att.reshape(nb, k, dh)
            m = jnp.max(a3, axis=1, keepdims=True)
            e = jnp.exp(a3 - m)
            s = e / jnp.sum(e, axis=1, keepdims=True)
            return jnp.sum(f.reshape(nb, k, dh) * s, axis=1)

        aggn = soft_agg(att1, fnv)
        aggx = soft_agg(att2, fxv)
        y = (jnp.dot(aggn, wm1_ref[...], preferred_element_type=jnp.float32)
             + jnp.dot(aggx, wm2_ref[...], preferred_element_type=jnp.float32)
             + bm_ref[...])
        o_ref[...] = jnp.where(y >= 0, y, 0.2 * y)

    wspec = lambda shape: pl.BlockSpec(shape, lambda i: (0, 0))
    return pl.pallas_call(
        body,
        grid=(r // nb,),
        in_specs=[
            pl.BlockSpec((nb * k, dh), lambda i: (i, 0)),
            pl.BlockSpec((nb * k, dh), lambda i: (i, 0)),
            wspec((dh, dh)), wspec((dh, dh)), wspec((dh, dh)), wspec((dh, dh)),
            wspec((1, dh)), wspec((1, dh)),
            wspec((dh, dmlp)), wspec((dh, dmlp)), wspec((1, dmlp)),
        ],
        out_specs=pl.BlockSpec((nb, dmlp), lambda i: (i, 0)),
        out_shape=jax.ShapeDtypeStruct((r, dmlp), jnp.float32),
    )(fn, fx, a1, b1, a2, b2,
      bf1.reshape(1, dh), bf2.reshape(1, dh), wm1, wm2, bm.reshape(1, dmlp))


def _maxpool_k(x, r_out):
    d = x.shape[1]
    k = _KNN
    nb = _blk(r_out, 256)

    def body(x_ref, o_ref):
        o_ref[...] = jnp.max(x_ref[...].reshape(nb, k, d), axis=1)

    return pl.pallas_call(
        body,
        grid=(r_out // nb,),
        in_specs=[pl.BlockSpec((nb * k, d), lambda i: (i, 0))],
        out_specs=pl.BlockSpec((nb, d), lambda i: (i, 0)),
        out_shape=jax.ShapeDtypeStruct((r_out, d), jnp.float32),
    )(x)


def _fold_bn(p):
    s = p['gamma'] * lax.rsqrt(p['var'] + 1e-6)
    return p['W'] * s[None, :], (p['b'] - p['mean']) * s + p['beta']


def _split_att(pfc, pmlp):
    w, b = pfc['W'], pfc['b']
    d = w.shape[0]
    dh = d // 2
    wm, bm = _fold_bn(pmlp)
    return dict(
        a1=w[:dh, :dh], b1=w[dh:, :dh], a2=w[:dh, dh:], b2=w[dh:, dh:],
        bf1=b[:dh], bf2=b[dh:], wm1=wm[:dh], wm2=wm[dh:], bm=bm,
    )


def kernel(features, xyz_0, xyz_1, xyz_2, xyz_3,
           neigh_idx_0, neigh_idx_1, neigh_idx_2, neigh_idx_3,
           sub_idx_0, sub_idx_1, sub_idx_2, sub_idx_3,
           interp_idx_0, interp_idx_1, interp_idx_2, interp_idx_3, params):
    P = params
    B = features.shape[0]
    xyzs = [xyz_0, xyz_1, xyz_2, xyz_3]
    neighs = [neigh_idx_0, neigh_idx_1, neigh_idx_2, neigh_idx_3]
    subs = [sub_idx_0, sub_idx_1, sub_idx_2, sub_idx_3]
    interps = [interp_idx_0, interp_idx_1, interp_idx_2, interp_idx_3]

    def flat_idx(idx, n_src):
        offs = (jnp.arange(B, dtype=idx.dtype) * n_src)[:, None, None]
        return (idx + offs).reshape(-1)

    w0 = P['fc0']['W']
    bn = P['bn0']
    s0 = bn['gamma'] * lax.rsqrt(bn['var'] + 1e-6)
    w0f = w0 * s0[None, :]
    b0f = (P['fc0']['b'] - bn['mean']) * s0 + bn['beta']
    f = _dense_act(features.reshape(B * _NS[0], 6), w0f, b0f, True, 0.3)

    enc = []
    for i in range(4):
        nm = 'Encoder_layer_%d' % i
        n = _NS[i]
        n_next = _NS[i + 1]
        nidx = flat_idx(neighs[i], n)
        xyz2d = xyzs[i].reshape(B * n, 3)

        w, b = _fold_bn(P[nm + 'mlp1'])
        fpc = _dense_act(f, w, b, True, 0.2)

        nbr_xyz = _gather_rows(xyz2d, nidx)
        w, b = _fold_bn(P[nm + 'LFAmlp1'])
        fx1 = _relpos_mlp(xyz2d, nbr_xyz, w, b)

        fn1 = _gather_rows(fpc, nidx)
        ap1 = _split_att(P[nm + 'LFAatt_pooling_1fc'], P[nm + 'LFAatt_pooling_1mlp'])
        fagg = _att_pool(fn1, fx1, ap1['a1'], ap1['b1'], ap1['a2'], ap1['b2'],
                         ap1['bf1'], ap1['bf2'], ap1['wm1'], ap1['wm2'], ap1['bm'])

        w, b = _fold_bn(P[nm + 'LFAmlp2'])
        fx2 = _dense_act(fx1, w, b, True, 0.2)
        fn2 = _gather_rows(fagg, nidx)
        ap2 = _split_att(P[nm + 'LFAatt_pooling_2fc'], P[nm + 'LFAatt_pooling_2mlp'])
        flfa = _att_pool(fn2, fx2, ap2['a1'], ap2['b1'], ap2['a2'], ap2['b2'],
                         ap2['bf1'], ap2['bf2'], ap2['wm1'], ap2['wm2'], ap2['bm'])

        w2, b2 = _fold_bn(P[nm + 'mlp2'])
        ws, bs = _fold_bn(P[nm + 'shortcut'])
        fe = _dense2_act(flfa, f, w2, ws, b2 + bs, True, 0.2)

        sidx = flat_idx(subs[i], n)
        fp = _gather_rows(fe, sidx)
        f = _maxpool_k(fp, B * n_next)

        if i == 0:
            enc.append(fe)
        enc.append(f)

    w, b = _fold_bn(P['decoder_0'])
    f = _dense_act(f, w, b, True, 0.2)

    for j in range(4):
        lev = 3 - j
        n_dst = _NS[lev]
        n_src = _NS[lev + 1]
        iidx = flat_idx(interps[lev][:, :, None, 0], n_src)
        fi = _gather_rows(f, iidx)
        w, b = _fold_bn(P['Decoder_layer_%d' % j])
        skip = enc[-j - 2]
        dskip = skip.shape[1]
        f = _dense2_act(skip, fi, w[:dskip], w[dskip:], b, True, 0.2)

    w, b = _fold_bn(P['fc1'])
    f = _dense_act(f, w, b, True, 0.2)
    w, b = _fold_bn(P['fc2'])
    f = _dense_act(f, w, b, True, 0.2)
    f = _dense_act(f, P['fc']['W'], P['fc']['b'], False, 0.0)
    return f.reshape(B, _NS[0], 13)


# speedup vs baseline: 9.0172x; 2.8963x over previous
import functools
import math

import jax
import jax.numpy as jnp
from jax import lax
from jax.experimental import pallas as pl
from jax.experimental.pallas import tpu as pltpu
from jax.experimental.pallas import tpu_sc as plsc

_KNN = 16
_NS = [40960, 10240, 2560, 640, 160]
_DOUT = [16, 64, 128, 256]
_BATCH = 2


_NC = 2
_NSC = 16
_NW = _NC * _NSC


def _sc_gather_fn(m_pad, t, d):
    rows_pw = m_pad // _NW
    tot = rows_pw // 128
    ch_cap = max(1, min(16, (380 * 1024) // (128 * d * 4)))
    ch = 1
    for c in range(min(ch_cap, tot), 0, -1):
        if tot % c == 0:
            ch = c
            break
    iters = tot // ch
    mesh = plsc.VectorSubcoreMesh(core_axis_name="c", subcore_axis_name="s")

    @functools.partial(
        pl.kernel, mesh=mesh,
        out_type=jax.ShapeDtypeStruct((m_pad, d), jnp.float32),
        compiler_params=pltpu.CompilerParams(use_tc_tiling_on_sc=False),
        scratch_types=[
            pltpu.VMEM((ch, 128), jnp.int32),
            pltpu.VMEM((ch * 128, d), jnp.float32),
            pltpu.SemaphoreType.DMA,
        ],
    )
    def k(table_hbm, idx_hbm, out_hbm, idx_v, rows_v, sem):
        wid = lax.axis_index("s") * _NC + lax.axis_index("c")

        def body(it, carry):
            grp = wid * tot + it * ch
            pltpu.sync_copy(idx_hbm.at[pl.ds(grp, ch)], idx_v)
            cps = []
            for j in range(ch):
                cp = pltpu.make_async_copy(
                    table_hbm.at[idx_v.at[j]],
                    rows_v.at[pl.ds(j * 128, 128)], sem)
                cp.start()
                cps.append(cp)
            for cp in cps:
                cp.wait()
            pltpu.sync_copy(rows_v, out_hbm.at[pl.ds(grp * 128, ch * 128)])
            return carry

        lax.fori_loop(0, iters, body, 0, unroll=False)

    return k


def _gather_rows(table, idx):
    t, d = table.shape
    m = idx.shape[0]
    m_pad = -(-m // (_NW * 128)) * (_NW * 128)
    if m_pad != m:
        idx = jnp.pad(idx, (0, m_pad - m))
    out = _sc_gather_fn(m_pad, t, d)(table, idx.reshape(m_pad // 128, 128))
    return out if m_pad == m else out[:m]


def _blk(rows, cap):
    return math.gcd(rows, cap)


def _dense_act(x, w, b, act, slope):
    r, din = x.shape
    dout = w.shape[1]
    nb = _blk(r, 512)

    def body(x_ref, w_ref, b_ref, o_ref):
        y = jnp.dot(x_ref[...], w_ref[...],
                    preferred_element_type=jnp.float32) + b_ref[...]
        if act:
            y = jnp.where(y >= 0, y, slope * y)
        o_ref[...] = y

    return pl.pallas_call(
        body,
        grid=(r // nb,),
        in_specs=[
            pl.BlockSpec((nb, din), lambda i: (i, 0)),
            pl.BlockSpec((din, dout), lambda i: (0, 0)),
            pl.BlockSpec((1, dout), lambda i: (0, 0)),
        ],
        out_specs=pl.BlockSpec((nb, dout), lambda i: (i, 0)),
        out_shape=jax.ShapeDtypeStruct((r, dout), jnp.float32),
    )(x, w, b.reshape(1, dout))


def _dense2_act(x1, x2, w1, w2, b, act, slope):
    r, d1 = x1.shape
    d2 = x2.shape[1]
    dout = w1.shape[1]
    nb = _blk(r, 512)

    def body(x1_ref, x2_ref, w1_ref, w2_ref, b_ref, o_ref):
        y = (jnp.dot(x1_ref[...], w1_ref[...], preferred_element_type=jnp.float32)
             + jnp.dot(x2_ref[...], w2_ref[...], preferred_element_type=jnp.float32)
             + b_ref[...])
        if act:
            y = jnp.where(y >= 0, y, slope * y)
        o_ref[...] = y

    return pl.pallas_call(
        body,
        grid=(r // nb,),
        in_specs=[
            pl.BlockSpec((nb, d1), lambda i: (i, 0)),
            pl.BlockSpec((nb, d2), lambda i: (i, 0)),
            pl.BlockSpec((d1, dout), lambda i: (0, 0)),
            pl.BlockSpec((d2, dout), lambda i: (0, 0)),
            pl.BlockSpec((1, dout), lambda i: (0, 0)),
        ],
        out_specs=pl.BlockSpec((nb, dout), lambda i: (i, 0)),
        out_shape=jax.ShapeDtypeStruct((r, dout), jnp.float32),
    )(x1, x2, w1, w2, b.reshape(1, dout))


def _relpos_mlp(xyz, nbr_xyz, w, b):
    r = xyz.shape[0]
    dh = w.shape[1]
    k = _KNN
    nb = _blk(r, 256)

    def body(xyz_ref, nbr_ref, w_ref, b_ref, o_ref):
        tile = xyz_ref[...]
        tile = jnp.broadcast_to(tile[:, None, :], (nb, k, 3)).reshape(nb * k, 3)
        nbr = nbr_ref[...]
        rel = tile - nbr
        dis = jnp.sqrt(jnp.sum(rel * rel, axis=-1, keepdims=True) + 1e-12)
        feat = jnp.concatenate([dis, rel, tile, nbr], axis=-1)
        y = jnp.dot(feat, w_ref[...], preferred_element_type=jnp.float32) + b_ref[...]
        o_ref[...] = jnp.where(y >= 0, y, 0.2 * y)

    return pl.pallas_call(
        body,
        grid=(r // nb,),
        in_specs=[
            pl.BlockSpec((nb, 3), lambda i: (i, 0)),
            pl.BlockSpec((nb * k, 3), lambda i: (i, 0)),
            pl.BlockSpec((10, dh), lambda i: (0, 0)),
            pl.BlockSpec((1, dh), lambda i: (0, 0)),
        ],
        out_specs=pl.BlockSpec((nb * k, dh), lambda i: (i, 0)),
        out_shape=jax.ShapeDtypeStruct((r * k, dh), jnp.float32),
    )(xyz, nbr_xyz, w, b.reshape(1, dh))


def _att_pool(fn, fx, a1, b1, a2, b2, bf1, bf2, wm1, wm2, bm):
    rk, dh = fn.shape
    k = _KNN
    r = rk // k
    dmlp = wm1.shape[1]
    nb = _blk(r, 256)

    def body(fn_ref, fx_ref, a1_ref, b1_ref, a2_ref, b2_ref,
             bf1_ref, bf2_ref, wm1_ref, wm2_ref, bm_ref, o_ref):
        fnv = fn_ref[...]
        fxv = fx_ref[...]
        att1 = (jnp.dot(fnv, a1_ref[...], preferred_element_type=jnp.float32)
                + jnp.dot(fxv, b1_ref[...], preferred_element_type=jnp.float32)
                + bf1_ref[...])
        att2 = (jnp.dot(fnv, a2_ref[...], preferred_element_type=jnp.float32)
                + jnp.dot(fxv, b2_ref[...], preferred_element_type=jnp.float32)
                + bf2_ref[...])

        def soft_agg(att, f):
            a3 = att.reshape(nb, k, dh)
            m = jnp.max(a3, axis=1, keepdims=True)
            e = jnp.exp(a3 - m)
            s = e / jnp.sum(e, axis=1, keepdims=True)
            return jnp.sum(f.reshape(nb, k, dh) * s, axis=1)

        aggn = soft_agg(att1, fnv)
        aggx = soft_agg(att2, fxv)
        y = (jnp.dot(aggn, wm1_ref[...], preferred_element_type=jnp.float32)
             + jnp.dot(aggx, wm2_ref[...], preferred_element_type=jnp.float32)
             + bm_ref[...])
        o_ref[...] = jnp.where(y >= 0, y, 0.2 * y)

    wspec = lambda shape: pl.BlockSpec(shape, lambda i: (0, 0))
    return pl.pallas_call(
        body,
        grid=(r // nb,),
        in_specs=[
            pl.BlockSpec((nb * k, dh), lambda i: (i, 0)),
            pl.BlockSpec((nb * k, dh), lambda i: (i, 0)),
            wspec((dh, dh)), wspec((dh, dh)), wspec((dh, dh)), wspec((dh, dh)),
            wspec((1, dh)), wspec((1, dh)),
            wspec((dh, dmlp)), wspec((dh, dmlp)), wspec((1, dmlp)),
        ],
        out_specs=pl.BlockSpec((nb, dmlp), lambda i: (i, 0)),
        out_shape=jax.ShapeDtypeStruct((r, dmlp), jnp.float32),
    )(fn, fx, a1, b1, a2, b2,
      bf1.reshape(1, dh), bf2.reshape(1, dh), wm1, wm2, bm.reshape(1, dmlp))


def _maxpool_k(x, r_out):
    d = x.shape[1]
    k = _KNN
    nb = _blk(r_out, 256)

    def body(x_ref, o_ref):
        o_ref[...] = jnp.max(x_ref[...].reshape(nb, k, d), axis=1)

    return pl.pallas_call(
        body,
        grid=(r_out // nb,),
        in_specs=[pl.BlockSpec((nb * k, d), lambda i: (i, 0))],
        out_specs=pl.BlockSpec((nb, d), lambda i: (i, 0)),
        out_shape=jax.ShapeDtypeStruct((r_out, d), jnp.float32),
    )(x)


def _fold_bn(p):
    s = p['gamma'] * lax.rsqrt(p['var'] + 1e-6)
    return p['W'] * s[None, :], (p['b'] - p['mean']) * s + p['beta']


def _split_att(pfc, pmlp):
    w, b = pfc['W'], pfc['b']
    d = w.shape[0]
    dh = d // 2
    wm, bm = _fold_bn(pmlp)
    return dict(
        a1=w[:dh, :dh], b1=w[dh:, :dh], a2=w[:dh, dh:], b2=w[dh:, dh:],
        bf1=b[:dh], bf2=b[dh:], wm1=wm[:dh], wm2=wm[dh:], bm=bm,
    )


def kernel(features, xyz_0, xyz_1, xyz_2, xyz_3,
           neigh_idx_0, neigh_idx_1, neigh_idx_2, neigh_idx_3,
           sub_idx_0, sub_idx_1, sub_idx_2, sub_idx_3,
           interp_idx_0, interp_idx_1, interp_idx_2, interp_idx_3, params):
    P = params
    B = features.shape[0]
    xyzs = [xyz_0, xyz_1, xyz_2, xyz_3]
    neighs = [neigh_idx_0, neigh_idx_1, neigh_idx_2, neigh_idx_3]
    subs = [sub_idx_0, sub_idx_1, sub_idx_2, sub_idx_3]
    interps = [interp_idx_0, interp_idx_1, interp_idx_2, interp_idx_3]

    def flat_idx(idx, n_src):
        offs = (jnp.arange(B, dtype=idx.dtype) * n_src)[:, None, None]
        return (idx + offs).reshape(-1)

    w0 = P['fc0']['W']
    bn = P['bn0']
    s0 = bn['gamma'] * lax.rsqrt(bn['var'] + 1e-6)
    w0f = w0 * s0[None, :]
    b0f = (P['fc0']['b'] - bn['mean']) * s0 + bn['beta']
    f = _dense_act(features.reshape(B * _NS[0], 6), w0f, b0f, True, 0.3)

    enc = []
    for i in range(4):
        nm = 'Encoder_layer_%d' % i
        n = _NS[i]
        n_next = _NS[i + 1]
        nidx = flat_idx(neighs[i], n)
        xyz2d = xyzs[i].reshape(B * n, 3)

        w, b = _fold_bn(P[nm + 'mlp1'])
        fpc = _dense_act(f, w, b, True, 0.2)

        nbr_xyz = _gather_rows(xyz2d, nidx)
        w, b = _fold_bn(P[nm + 'LFAmlp1'])
        fx1 = _relpos_mlp(xyz2d, nbr_xyz, w, b)

        fn1 = _gather_rows(fpc, nidx)
        ap1 = _split_att(P[nm + 'LFAatt_pooling_1fc'], P[nm + 'LFAatt_pooling_1mlp'])
        fagg = _att_pool(fn1, fx1, ap1['a1'], ap1['b1'], ap1['a2'], ap1['b2'],
                         ap1['bf1'], ap1['bf2'], ap1['wm1'], ap1['wm2'], ap1['bm'])

        w, b = _fold_bn(P[nm + 'LFAmlp2'])
        fx2 = _dense_act(fx1, w, b, True, 0.2)
        fn2 = _gather_rows(fagg, nidx)
        ap2 = _split_att(P[nm + 'LFAatt_pooling_2fc'], P[nm + 'LFAatt_pooling_2mlp'])
        flfa = _att_pool(fn2, fx2, ap2['a1'], ap2['b1'], ap2['a2'], ap2['b2'],
                         ap2['bf1'], ap2['bf2'], ap2['wm1'], ap2['wm2'], ap2['bm'])

        w2, b2 = _fold_bn(P[nm + 'mlp2'])
        ws, bs = _fold_bn(P[nm + 'shortcut'])
        fe = _dense2_act(flfa, f, w2, ws, b2 + bs, True, 0.2)

        sidx = flat_idx(subs[i], n)
        fp = _gather_rows(fe, sidx)
        f = _maxpool_k(fp, B * n_next)

        if i == 0:
            enc.append(fe)
        enc.append(f)

    w, b = _fold_bn(P['decoder_0'])
    f = _dense_act(f, w, b, True, 0.2)

    for j in range(4):
        lev = 3 - j
        n_dst = _NS[lev]
        n_src = _NS[lev + 1]
        iidx = flat_idx(interps[lev][:, :, None, 0], n_src)
        fi = _gather_rows(f, iidx)
        w, b = _fold_bn(P['Decoder_layer_%d' % j])
        skip = enc[-j - 2]
        dskip = skip.shape[1]
        f = _dense2_act(skip, fi, w[:dskip], w[dskip:], b, True, 0.2)

    w, b = _fold_bn(P['fc1'])
    f = _dense_act(f, w, b, True, 0.2)
    w, b = _fold_bn(P['fc2'])
    f = _dense_act(f, w, b, True, 0.2)
    f = _dense_act(f, P['fc']['W'], P['fc']['b'], False, 0.0)
    return f.reshape(B, _NS[0], 13)


# speedup vs baseline: 11.9760x; 1.3281x over previous
import functools
import math

import jax
import jax.numpy as jnp
from jax import lax
from jax.experimental import pallas as pl
from jax.experimental.pallas import tpu as pltpu
from jax.experimental.pallas import tpu_sc as plsc

_KNN = 16
_NS = [40960, 10240, 2560, 640, 160]
_DOUT = [16, 64, 128, 256]
_BATCH = 2


_NC = 2
_NSC = 16
_NW = _NC * _NSC


def _sc_gather_fn(m_pad, t, d):
    rows_pw = m_pad // _NW
    tot = rows_pw // 128
    ch_cap = max(1, min(16, (380 * 1024) // (128 * d * 4)))
    ch = 1
    for c in range(min(ch_cap, tot), 0, -1):
        if tot % c == 0:
            ch = c
            break
    iters = tot // ch
    mesh = plsc.VectorSubcoreMesh(core_axis_name="c", subcore_axis_name="s")

    @functools.partial(
        pl.kernel, mesh=mesh,
        out_type=jax.ShapeDtypeStruct((m_pad, d), jnp.float32),
        compiler_params=pltpu.CompilerParams(use_tc_tiling_on_sc=False),
        scratch_types=[
            pltpu.VMEM((ch, 128), jnp.int32),
            pltpu.VMEM((ch * 128, d), jnp.float32),
            pltpu.SemaphoreType.DMA,
        ],
    )
    def k(table_hbm, idx_hbm, out_hbm, idx_v, rows_v, sem):
        wid = lax.axis_index("s") * _NC + lax.axis_index("c")

        def body(it, carry):
            grp = wid * tot + it * ch
            pltpu.sync_copy(idx_hbm.at[pl.ds(grp, ch)], idx_v)
            cps = []
            for j in range(ch):
                cp = pltpu.make_async_copy(
                    table_hbm.at[idx_v.at[j]],
                    rows_v.at[pl.ds(j * 128, 128)], sem)
                cp.start()
                cps.append(cp)
            for cp in cps:
                cp.wait()
            pltpu.sync_copy(rows_v, out_hbm.at[pl.ds(grp * 128, ch * 128)])
            return carry

        lax.fori_loop(0, iters, body, 0, unroll=False)

    return k


def _gather_rows(table, idx):
    t, d = table.shape
    m = idx.shape[0]
    m_pad = -(-m // (_NW * 128)) * (_NW * 128)
    if m_pad != m:
        idx = jnp.pad(idx, (0, m_pad - m))
    out = _sc_gather_fn(m_pad, t, d)(table, idx.reshape(m_pad // 128, 128))
    return out if m_pad == m else out[:m]


def _blk(rows, cap):
    return math.gcd(rows, cap)


def _blk_for(rows, width, target_bytes=2 * 1024 * 1024, cap=4096):
    want = max(128, min(cap, target_bytes // max(1, 4 * width)))
    p = 1 << (want.bit_length() - 1)
    return math.gcd(rows, p)


def _dense_act(x, w, b, act, slope):
    r, din = x.shape
    dout = w.shape[1]
    nb = _blk_for(r, max(din, dout))

    def body(x_ref, w_ref, b_ref, o_ref):
        y = jnp.dot(x_ref[...], w_ref[...],
                    preferred_element_type=jnp.float32) + b_ref[...]
        if act:
            y = jnp.where(y >= 0, y, slope * y)
        o_ref[...] = y

    return pl.pallas_call(
        body,
        grid=(r // nb,),
        in_specs=[
            pl.BlockSpec((nb, din), lambda i: (i, 0)),
            pl.BlockSpec((din, dout), lambda i: (0, 0)),
            pl.BlockSpec((1, dout), lambda i: (0, 0)),
        ],
        out_specs=pl.BlockSpec((nb, dout), lambda i: (i, 0)),
        out_shape=jax.ShapeDtypeStruct((r, dout), jnp.float32),
    )(x, w, b.reshape(1, dout))


def _dense2_act(x1, x2, w1, w2, b, act, slope):
    r, d1 = x1.shape
    d2 = x2.shape[1]
    dout = w1.shape[1]
    nb = _blk_for(r, max(d1 + d2, dout))

    def body(x1_ref, x2_ref, w1_ref, w2_ref, b_ref, o_ref):
        y = (jnp.dot(x1_ref[...], w1_ref[...], preferred_element_type=jnp.float32)
             + jnp.dot(x2_ref[...], w2_ref[...], preferred_element_type=jnp.float32)
             + b_ref[...])
        if act:
            y = jnp.where(y >= 0, y, slope * y)
        o_ref[...] = y

    return pl.pallas_call(
        body,
        grid=(r // nb,),
        in_specs=[
            pl.BlockSpec((nb, d1), lambda i: (i, 0)),
            pl.BlockSpec((nb, d2), lambda i: (i, 0)),
            pl.BlockSpec((d1, dout), lambda i: (0, 0)),
            pl.BlockSpec((d2, dout), lambda i: (0, 0)),
            pl.BlockSpec((1, dout), lambda i: (0, 0)),
        ],
        out_specs=pl.BlockSpec((nb, dout), lambda i: (i, 0)),
        out_shape=jax.ShapeDtypeStruct((r, dout), jnp.float32),
    )(x1, x2, w1, w2, b.reshape(1, dout))


def _mlp1_pack(x, xyz, w, b, d16):
    r, din = x.shape
    dh = w.shape[1]
    pad = d16 - dh - 3
    nb = _blk_for(r, max(din, d16))

    def body(x_ref, xyz_ref, w_ref, b_ref, o_ref):
        y = jnp.dot(x_ref[...], w_ref[...],
                    preferred_element_type=jnp.float32) + b_ref[...]
        y = jnp.where(y >= 0, y, 0.2 * y)
        o_ref[...] = jnp.concatenate(
            [y, xyz_ref[...], jnp.zeros((nb, pad), jnp.float32)], axis=-1)

    return pl.pallas_call(
        body,
        grid=(r // nb,),
        in_specs=[
            pl.BlockSpec((nb, din), lambda i: (i, 0)),
            pl.BlockSpec((nb, 3), lambda i: (i, 0)),
            pl.BlockSpec((din, dh), lambda i: (0, 0)),
            pl.BlockSpec((1, dh), lambda i: (0, 0)),
        ],
        out_specs=pl.BlockSpec((nb, d16), lambda i: (i, 0)),
        out_shape=jax.ShapeDtypeStruct((r, d16), jnp.float32),
    )(x, xyz, w, b.reshape(1, dh))


def _relpos_mlp(xyz, gat, nbr_off, w, b):
    r = xyz.shape[0]
    d16 = gat.shape[1]
    dh = w.shape[1]
    k = _KNN
    nb = _blk_for(r, _KNN * (d16 + dh), cap=2048)

    def body(xyz_ref, nbr_ref, w_ref, b_ref, o_ref):
        tile = xyz_ref[...]
        tile = jnp.broadcast_to(tile[:, None, :], (nb, k, 3)).reshape(nb * k, 3)
        nbr = nbr_ref[...][:, nbr_off:nbr_off + 3]
        rel = tile - nbr
        dis = jnp.sqrt(jnp.sum(rel * rel, axis=-1, keepdims=True) + 1e-12)
        feat = jnp.concatenate([dis, rel, tile, nbr], axis=-1)
        y = jnp.dot(feat, w_ref[...], preferred_element_type=jnp.float32) + b_ref[...]
        o_ref[...] = jnp.where(y >= 0, y, 0.2 * y)

    return pl.pallas_call(
        body,
        grid=(r // nb,),
        in_specs=[
            pl.BlockSpec((nb, 3), lambda i: (i, 0)),
            pl.BlockSpec((nb * k, d16), lambda i: (i, 0)),
            pl.BlockSpec((10, dh), lambda i: (0, 0)),
            pl.BlockSpec((1, dh), lambda i: (0, 0)),
        ],
        out_specs=pl.BlockSpec((nb * k, dh), lambda i: (i, 0)),
        out_shape=jax.ShapeDtypeStruct((r * k, dh), jnp.float32),
    )(xyz, gat, w, b.reshape(1, dh))


def _att_pool(fn, fx, a1, b1, a2, b2, bf1, bf2, wm1, wm2, bm):
    rk, dfn = fn.shape
    dh = fx.shape[1]
    k = _KNN
    r = rk // k
    dmlp = wm1.shape[1]
    nb = _blk_for(r, _KNN * (dfn + 3 * dh), cap=2048)

    def body(fn_ref, fx_ref, a1_ref, b1_ref, a2_ref, b2_ref,
             bf1_ref, bf2_ref, wm1_ref, wm2_ref, bm_ref, o_ref):
        fnv = fn_ref[...][:, :dh]
        fxv = fx_ref[...]
        att1 = (jnp.dot(fnv, a1_ref[...], preferred_element_type=jnp.float32)
                + jnp.dot(fxv, b1_ref[...], preferred_element_type=jnp.float32)
                + bf1_ref[...])
        att2 = (jnp.dot(fnv, a2_ref[...], preferred_element_type=jnp.float32)
                + jnp.dot(fxv, b2_ref[...], preferred_element_type=jnp.float32)
                + bf2_ref[...])

        def soft_agg(att, f):
            a3 = att.reshape(nb, k, dh)
            m = jnp.max(a3, axis=1, keepdims=True)
            e = jnp.exp(a3 - m)
            s = e / jnp.sum(e, axis=1, keepdims=True)
            return jnp.sum(f.reshape(nb, k, dh) * s, axis=1)

        aggn = soft_agg(att1, fnv)
        aggx = soft_agg(att2, fxv)
        y = (jnp.dot(aggn, wm1_ref[...], preferred_element_type=jnp.float32)
             + jnp.dot(aggx, wm2_ref[...], preferred_element_type=jnp.float32)
             + bm_ref[...])
        o_ref[...] = jnp.where(y >= 0, y, 0.2 * y)

    wspec = lambda shape: pl.BlockSpec(shape, lambda i: (0, 0))
    return pl.pallas_call(
        body,
        grid=(r // nb,),
        in_specs=[
            pl.BlockSpec((nb * k, dfn), lambda i: (i, 0)),
            pl.BlockSpec((nb * k, dh), lambda i: (i, 0)),
            wspec((dh, dh)), wspec((dh, dh)), wspec((dh, dh)), wspec((dh, dh)),
            wspec((1, dh)), wspec((1, dh)),
            wspec((dh, dmlp)), wspec((dh, dmlp)), wspec((1, dmlp)),
        ],
        out_specs=pl.BlockSpec((nb, dmlp), lambda i: (i, 0)),
        out_shape=jax.ShapeDtypeStruct((r, dmlp), jnp.float32),
    )(fn, fx, a1, b1, a2, b2,
      bf1.reshape(1, dh), bf2.reshape(1, dh), wm1, wm2, bm.reshape(1, dmlp))


def _maxpool_k(x, r_out):
    d = x.shape[1]
    k = _KNN
    nb = _blk_for(r_out, _KNN * d, cap=2048)

    def body(x_ref, o_ref):
        o_ref[...] = jnp.max(x_ref[...].reshape(nb, k, d), axis=1)

    return pl.pallas_call(
        body,
        grid=(r_out // nb,),
        in_specs=[pl.BlockSpec((nb * k, d), lambda i: (i, 0))],
        out_specs=pl.BlockSpec((nb, d), lambda i: (i, 0)),
        out_shape=jax.ShapeDtypeStruct((r_out, d), jnp.float32),
    )(x)


def _fold_bn(p):
    s = p['gamma'] * lax.rsqrt(p['var'] + 1e-6)
    return p['W'] * s[None, :], (p['b'] - p['mean']) * s + p['beta']


def _split_att(pfc, pmlp):
    w, b = pfc['W'], pfc['b']
    d = w.shape[0]
    dh = d // 2
    wm, bm = _fold_bn(pmlp)
    return dict(
        a1=w[:dh, :dh], b1=w[dh:, :dh], a2=w[:dh, dh:], b2=w[dh:, dh:],
        bf1=b[:dh], bf2=b[dh:], wm1=wm[:dh], wm2=wm[dh:], bm=bm,
    )


def kernel(features, xyz_0, xyz_1, xyz_2, xyz_3,
           neigh_idx_0, neigh_idx_1, neigh_idx_2, neigh_idx_3,
           sub_idx_0, sub_idx_1, sub_idx_2, sub_idx_3,
           interp_idx_0, interp_idx_1, interp_idx_2, interp_idx_3, params):
    P = params
    B = features.shape[0]
    xyzs = [xyz_0, xyz_1, xyz_2, xyz_3]
    neighs = [neigh_idx_0, neigh_idx_1, neigh_idx_2, neigh_idx_3]
    subs = [sub_idx_0, sub_idx_1, sub_idx_2, sub_idx_3]
    interps = [interp_idx_0, interp_idx_1, interp_idx_2, interp_idx_3]

    def flat_idx(idx, n_src):
        offs = (jnp.arange(B, dtype=idx.dtype) * n_src)[:, None, None]
        return (idx + offs).reshape(-1)

    w0 = P['fc0']['W']
    bn = P['bn0']
    s0 = bn['gamma'] * lax.rsqrt(bn['var'] + 1e-6)
    w0f = w0 * s0[None, :]
    b0f = (P['fc0']['b'] - bn['mean']) * s0 + bn['beta']
    f = _dense_act(features.reshape(B * _NS[0], 6), w0f, b0f, True, 0.3)

    enc = []
    for i in range(4):
        nm = 'Encoder_layer_%d' % i
        n = _NS[i]
        n_next = _NS[i + 1]
        nidx = flat_idx(neighs[i], n)
        xyz2d = xyzs[i].reshape(B * n, 3)

        w, b = _fold_bn(P[nm + 'mlp1'])
        dh = w.shape[1]
        d16 = -(-(dh + 3) // 8) * 8
        tbl1 = _mlp1_pack(f, xyz2d, w, b, d16)
        g1 = _gather_rows(tbl1, nidx)

        w, b = _fold_bn(P[nm + 'LFAmlp1'])
        fx1 = _relpos_mlp(xyz2d, g1, dh, w, b)

        ap1 = _split_att(P[nm + 'LFAatt_pooling_1fc'], P[nm + 'LFAatt_pooling_1mlp'])
        fagg = _att_pool(g1, fx1, ap1['a1'], ap1['b1'], ap1['a2'], ap1['b2'],
                         ap1['bf1'], ap1['bf2'], ap1['wm1'], ap1['wm2'], ap1['bm'])

        w, b = _fold_bn(P[nm + 'LFAmlp2'])
        fx2 = _dense_act(fx1, w, b, True, 0.2)
        fn2 = _gather_rows(fagg, nidx)
        ap2 = _split_att(P[nm + 'LFAatt_pooling_2fc'], P[nm + 'LFAatt_pooling_2mlp'])
        flfa = _att_pool(fn2, fx2, ap2['a1'], ap2['b1'], ap2['a2'], ap2['b2'],
                         ap2['bf1'], ap2['bf2'], ap2['wm1'], ap2['wm2'], ap2['bm'])

        w2, b2 = _fold_bn(P[nm + 'mlp2'])
        ws, bs = _fold_bn(P[nm + 'shortcut'])
        fe = _dense2_act(flfa, f, w2, ws, b2 + bs, True, 0.2)


        sidx = flat_idx(subs[i], n)
        fp = _gather_rows(fe, sidx)
        f = _maxpool_k(fp, B * n_next)

        if i == 0:
            enc.append(fe)
        enc.append(f)

    w, b = _fold_bn(P['decoder_0'])
    f = _dense_act(f, w, b, True, 0.2)

    for j in range(4):
        lev = 3 - j
        n_dst = _NS[lev]
        n_src = _NS[lev + 1]
        iidx = flat_idx(interps[lev][:, :, None, 0], n_src)
        fi = _gather_rows(f, iidx)
        w, b = _fold_bn(P['Decoder_layer_%d' % j])
        skip = enc[-j - 2]
        dskip = skip.shape[1]
        f = _dense2_act(skip, fi, w[:dskip], w[dskip:], b, True, 0.2)

    w, b = _fold_bn(P['fc1'])
    f = _dense_act(f, w, b, True, 0.2)
    w, b = _fold_bn(P['fc2'])
    f = _dense_act(f, w, b, True, 0.2)
    f = _dense_act(f, P['fc']['W'], P['fc']['b'], False, 0.0)
    return f.reshape(B, _NS[0], 13)


# speedup vs baseline: 26.3963x; 2.2041x over previous
import functools
import math

import jax
import jax.numpy as jnp
from jax import lax
from jax.experimental import pallas as pl
from jax.experimental.pallas import tpu as pltpu
from jax.experimental.pallas import tpu_sc as plsc

_KNN = 16
_NS = [40960, 10240, 2560, 640, 160]
_DOUT = [16, 64, 128, 256]
_BATCH = 2


_NC = 2
_NSC = 16
_NW = _NC * _NSC


def _sc_gather_fn(m_pad, t, d):
    rows_pw = m_pad // _NW
    tot = rows_pw // 128
    ch_cap = max(1, min(16, (380 * 1024) // (128 * d * 4)))
    ch = 1
    for c in range(min(ch_cap, tot), 0, -1):
        if tot % c == 0:
            ch = c
            break
    iters = tot // ch
    mesh = plsc.VectorSubcoreMesh(core_axis_name="c", subcore_axis_name="s")

    @functools.partial(
        pl.kernel, mesh=mesh,
        out_type=jax.ShapeDtypeStruct((m_pad, d), jnp.float32),
        compiler_params=pltpu.CompilerParams(use_tc_tiling_on_sc=False),
        scratch_types=[
            pltpu.VMEM((ch, 128), jnp.int32),
            pltpu.VMEM((ch * 128, d), jnp.float32),
            pltpu.SemaphoreType.DMA,
        ],
    )
    def k(table_hbm, idx_hbm, out_hbm, idx_v, rows_v, sem):
        wid = lax.axis_index("s") * _NC + lax.axis_index("c")

        def body(it, carry):
            grp = wid * tot + it * ch
            pltpu.sync_copy(idx_hbm.at[pl.ds(grp, ch)], idx_v)
            cps = []
            for j in range(ch):
                cp = pltpu.make_async_copy(
                    table_hbm.at[idx_v.at[j]],
                    rows_v.at[pl.ds(j * 128, 128)], sem)
                cp.start()
                cps.append(cp)
            for cp in cps:
                cp.wait()
            pltpu.sync_copy(rows_v, out_hbm.at[pl.ds(grp * 128, ch * 128)])
            return carry

        lax.fori_loop(0, iters, body, 0, unroll=False)

    return k


def _gather_rows(table, idx):
    t, d = table.shape
    m = idx.shape[0]
    m_pad = -(-m // (_NW * 128)) * (_NW * 128)
    if m_pad != m:
        idx = jnp.pad(idx, (0, m_pad - m))
    out = _sc_gather_fn(m_pad, t, d)(table, idx.reshape(m_pad // 128, 128))
    return out if m_pad == m else out[:m]


def _blk(rows, cap):
    return math.gcd(rows, cap)


def _blk_for(rows, width, target_bytes=2 * 1024 * 1024, cap=4096):
    want = max(128, min(cap, target_bytes // max(1, 4 * width)))
    p = 1 << (want.bit_length() - 1)
    return math.gcd(rows, p)


def _dense_act(x, w, b, act, slope):
    r, din = x.shape
    dout = w.shape[1]
    nb = _blk_for(r, max(din, dout))

    def body(x_ref, w_ref, b_ref, o_ref):
        y = jnp.dot(x_ref[...], w_ref[...],
                    preferred_element_type=jnp.float32) + b_ref[...]
        if act:
            y = jnp.where(y >= 0, y, slope * y)
        o_ref[...] = y

    return pl.pallas_call(
        body,
        grid=(r // nb,),
        in_specs=[
            pl.BlockSpec((nb, din), lambda i: (i, 0)),
            pl.BlockSpec((din, dout), lambda i: (0, 0)),
            pl.BlockSpec((1, dout), lambda i: (0, 0)),
        ],
        out_specs=pl.BlockSpec((nb, dout), lambda i: (i, 0)),
        out_shape=jax.ShapeDtypeStruct((r, dout), jnp.float32),
    )(x, w, b.reshape(1, dout))


def _dense2_act(x1, x2, w1, w2, b, act, slope):
    r, d1 = x1.shape
    d2 = x2.shape[1]
    dout = w1.shape[1]
    nb = _blk_for(r, max(d1 + d2, dout))

    def body(x1_ref, x2_ref, w1_ref, w2_ref, b_ref, o_ref):
        y = (jnp.dot(x1_ref[...], w1_ref[...], preferred_element_type=jnp.float32)
             + jnp.dot(x2_ref[...], w2_ref[...], preferred_element_type=jnp.float32)
             + b_ref[...])
        if act:
            y = jnp.where(y >= 0, y, slope * y)
        o_ref[...] = y

    return pl.pallas_call(
        body,
        grid=(r // nb,),
        in_specs=[
            pl.BlockSpec((nb, d1), lambda i: (i, 0)),
            pl.BlockSpec((nb, d2), lambda i: (i, 0)),
            pl.BlockSpec((d1, dout), lambda i: (0, 0)),
            pl.BlockSpec((d2, dout), lambda i: (0, 0)),
            pl.BlockSpec((1, dout), lambda i: (0, 0)),
        ],
        out_specs=pl.BlockSpec((nb, dout), lambda i: (i, 0)),
        out_shape=jax.ShapeDtypeStruct((r, dout), jnp.float32),
    )(x1, x2, w1, w2, b.reshape(1, dout))


def _mlp1_pack(x, xyz, w, b, d16):
    r, din = x.shape
    dh = w.shape[1]
    pad = d16 - dh - 3
    nb = _blk_for(r, max(din, d16))

    def body(x_ref, xyz_ref, w_ref, b_ref, o_ref):
        y = jnp.dot(x_ref[...], w_ref[...],
                    preferred_element_type=jnp.float32) + b_ref[...]
        y = jnp.where(y >= 0, y, 0.2 * y)
        o_ref[...] = jnp.concatenate(
            [y, xyz_ref[...], jnp.zeros((nb, pad), jnp.float32)], axis=-1)

    return pl.pallas_call(
        body,
        grid=(r // nb,),
        in_specs=[
            pl.BlockSpec((nb, din), lambda i: (i, 0)),
            pl.BlockSpec((nb, 3), lambda i: (i, 0)),
            pl.BlockSpec((din, dh), lambda i: (0, 0)),
            pl.BlockSpec((1, dh), lambda i: (0, 0)),
        ],
        out_specs=pl.BlockSpec((nb, d16), lambda i: (i, 0)),
        out_shape=jax.ShapeDtypeStruct((r, d16), jnp.float32),
    )(x, xyz, w, b.reshape(1, dh))


def _relpos_mlp(xyz, gat, nbr_off, w, b):
    r = xyz.shape[0]
    d16 = gat.shape[1]
    dh = w.shape[1]
    k = _KNN
    nb = _blk_for(r, _KNN * (d16 + dh), cap=2048)

    def body(xyz_ref, nbr_ref, w_ref, b_ref, o_ref):
        tile = xyz_ref[...]
        tile = jnp.broadcast_to(tile[:, None, :], (nb, k, 3)).reshape(nb * k, 3)
        nbr = nbr_ref[...][:, nbr_off:nbr_off + 3]
        rel = tile - nbr
        dis = jnp.sqrt(jnp.sum(rel * rel, axis=-1, keepdims=True) + 1e-12)
        feat = jnp.concatenate([dis, rel, tile, nbr], axis=-1)
        y = jnp.dot(feat, w_ref[...], preferred_element_type=jnp.float32) + b_ref[...]
        o_ref[...] = jnp.where(y >= 0, y, 0.2 * y)

    return pl.pallas_call(
        body,
        grid=(r // nb,),
        in_specs=[
            pl.BlockSpec((nb, 3), lambda i: (i, 0)),
            pl.BlockSpec((nb * k, d16), lambda i: (i, 0)),
            pl.BlockSpec((10, dh), lambda i: (0, 0)),
            pl.BlockSpec((1, dh), lambda i: (0, 0)),
        ],
        out_specs=pl.BlockSpec((nb * k, dh), lambda i: (i, 0)),
        out_shape=jax.ShapeDtypeStruct((r * k, dh), jnp.float32),
    )(xyz, gat, w, b.reshape(1, dh))


def _att_pool(fn, fx, a1, b1, a2, b2, bf1, bf2, wm1, wm2, bm):
    rk, dfn = fn.shape
    dh = fx.shape[1]
    k = _KNN
    r = rk // k
    dmlp = wm1.shape[1]
    nb = _blk_for(r, _KNN * (dfn + 3 * dh), cap=2048)

    def body(fn_ref, fx_ref, a1_ref, b1_ref, a2_ref, b2_ref,
             bf1_ref, bf2_ref, wm1_ref, wm2_ref, bm_ref, o_ref):
        fnv = fn_ref[...][:, :dh]
        fxv = fx_ref[...]
        att1 = (jnp.dot(fnv, a1_ref[...], preferred_element_type=jnp.float32)
                + jnp.dot(fxv, b1_ref[...], preferred_element_type=jnp.float32)
                + bf1_ref[...])
        att2 = (jnp.dot(fnv, a2_ref[...], preferred_element_type=jnp.float32)
                + jnp.dot(fxv, b2_ref[...], preferred_element_type=jnp.float32)
                + bf2_ref[...])

        def soft_agg(att, f):
            a3 = att.reshape(nb, k, dh)
            m = jnp.max(a3, axis=1, keepdims=True)
            e = jnp.exp(a3 - m)
            s = e / jnp.sum(e, axis=1, keepdims=True)
            return jnp.sum(f.reshape(nb, k, dh) * s, axis=1)

        aggn = soft_agg(att1, fnv)
        aggx = soft_agg(att2, fxv)
        y = (jnp.dot(aggn, wm1_ref[...], preferred_element_type=jnp.float32)
             + jnp.dot(aggx, wm2_ref[...], preferred_element_type=jnp.float32)
             + bm_ref[...])
        o_ref[...] = jnp.where(y >= 0, y, 0.2 * y)

    wspec = lambda shape: pl.BlockSpec(shape, lambda i: (0, 0))
    return pl.pallas_call(
        body,
        grid=(r // nb,),
        in_specs=[
            pl.BlockSpec((nb * k, dfn), lambda i: (i, 0)),
            pl.BlockSpec((nb * k, dh), lambda i: (i, 0)),
            wspec((dh, dh)), wspec((dh, dh)), wspec((dh, dh)), wspec((dh, dh)),
            wspec((1, dh)), wspec((1, dh)),
            wspec((dh, dmlp)), wspec((dh, dmlp)), wspec((1, dmlp)),
        ],
        out_specs=pl.BlockSpec((nb, dmlp), lambda i: (i, 0)),
        out_shape=jax.ShapeDtypeStruct((r, dmlp), jnp.float32),
    )(fn, fx, a1, b1, a2, b2,
      bf1.reshape(1, dh), bf2.reshape(1, dh), wm1, wm2, bm.reshape(1, dmlp))


def _maxpool_k(x, r_out):
    d = x.shape[1]
    k = _KNN
    nb = _blk_for(r_out, _KNN * d, cap=2048)

    def body(x_ref, o_ref):
        o_ref[...] = jnp.max(x_ref[...].reshape(nb, k, d), axis=1)

    return pl.pallas_call(
        body,
        grid=(r_out // nb,),
        in_specs=[pl.BlockSpec((nb * k, d), lambda i: (i, 0))],
        out_specs=pl.BlockSpec((nb, d), lambda i: (i, 0)),
        out_shape=jax.ShapeDtypeStruct((r_out, d), jnp.float32),
    )(x)


def _relpos_packed(xyz, gnp, sel_xyz, tile3, sum3, w0, w13, w46, w79, bt):
    r, kdfn = gnp.shape
    kdh = w13.shape[1]
    k = _KNN
    nb = _blk_for(r, kdfn + 3 * kdh, cap=2048)

    def body(xyz_ref, g_ref, sx_ref, t3_ref, s3_ref, w0_ref, w13_ref,
             w46_ref, w79_ref, bt_ref, o_ref):
        g = g_ref[...]
        dot = lambda a, b: jnp.dot(a, b, preferred_element_type=jnp.float32)
        nbr = dot(g, sx_ref[...])
        tile = dot(xyz_ref[...], t3_ref[...])
        rel = tile - nbr
        dis = jnp.sqrt(dot(rel * rel, s3_ref[...]) + 1e-12)
        y = (dot(dis, w0_ref[...]) + dot(rel, w13_ref[...])
             + dot(tile, w46_ref[...]) + dot(nbr, w79_ref[...]) + bt_ref[...])
        o_ref[...] = jnp.where(y >= 0, y, 0.2 * y)

    ws = lambda a: pl.BlockSpec(a.shape, lambda i: (0, 0))
    return pl.pallas_call(
        body,
        grid=(r // nb,),
        in_specs=[
            pl.BlockSpec((nb, 3), lambda i: (i, 0)),
            pl.BlockSpec((nb, kdfn), lambda i: (i, 0)),
            ws(sel_xyz), ws(tile3), ws(sum3), ws(w0), ws(w13), ws(w46),
            ws(w79), pl.BlockSpec((1, kdh), lambda i: (0, 0)),
        ],
        out_specs=pl.BlockSpec((nb, kdh), lambda i: (i, 0)),
        out_shape=jax.ShapeDtypeStruct((r, kdh), jnp.float32),
    )(xyz, gnp, sel_xyz, tile3, sum3, w0, w13, w46, w79, bt.reshape(1, kdh))


def _att_packed(gnp, fxp, a1bd, b1bd, a2bd, b2bd, sel_fn, ssum,
                bf1t, bf2t, wm1, wm2, bm):
    r, kdfn = gnp.shape
    kdh = fxp.shape[1]
    dh = ssum.shape[1]
    dmlp = wm1.shape[1]
    identity_sel = sel_fn is None
    nb = _blk_for(r, kdfn + 6 * kdh, cap=2048)

    def body(*refs):
        if identity_sel:
            (g_ref, fx_ref, a1_ref, b1_ref, a2_ref, b2_ref, ss_ref,
             bf1_ref, bf2_ref, wm1_ref, wm2_ref, bm_ref, o_ref) = refs
        else:
            (g_ref, fx_ref, a1_ref, b1_ref, a2_ref, b2_ref, sf_ref, ss_ref,
             bf1_ref, bf2_ref, wm1_ref, wm2_ref, bm_ref, o_ref) = refs
        dot = lambda a, b: jnp.dot(a, b, preferred_element_type=jnp.float32)
        g = g_ref[...]
        fx = fx_ref[...]
        att1 = dot(g, a1_ref[...]) + dot(fx, b1_ref[...]) + bf1_ref[...]
        att2 = dot(g, a2_ref[...]) + dot(fx, b2_ref[...]) + bf2_ref[...]
        e1 = jnp.exp(att1 - jnp.max(att1, axis=-1, keepdims=True))
        e2 = jnp.exp(att2 - jnp.max(att2, axis=-1, keepdims=True))
        fnp = g if identity_sel else dot(g, sf_ref[...])
        ss = ss_ref[...]
        aggn = dot(fnp * e1, ss) / dot(e1, ss)
        aggx = dot(fx * e2, ss) / dot(e2, ss)
        y = dot(aggn, wm1_ref[...]) + dot(aggx, wm2_ref[...]) + bm_ref[...]
        o_ref[...] = jnp.where(y >= 0, y, 0.2 * y)

    ws = lambda a: pl.BlockSpec(a.shape, lambda i: (0, 0))
    ins = [
        pl.BlockSpec((nb, kdfn), lambda i: (i, 0)),
        pl.BlockSpec((nb, kdh), lambda i: (i, 0)),
        ws(a1bd), ws(b1bd), ws(a2bd), ws(b2bd),
    ]
    args = [gnp, fxp, a1bd, b1bd, a2bd, b2bd]
    if not identity_sel:
        ins.append(ws(sel_fn))
        args.append(sel_fn)
    ins += [ws(ssum), pl.BlockSpec((1, kdh), lambda i: (0, 0)),
            pl.BlockSpec((1, kdh), lambda i: (0, 0)), ws(wm1), ws(wm2),
            pl.BlockSpec((1, dmlp), lambda i: (0, 0))]
    args += [ssum, bf1t.reshape(1, kdh), bf2t.reshape(1, kdh), wm1, wm2,
             bm.reshape(1, dmlp)]
    return pl.pallas_call(
        body,
        grid=(r // nb,),
        in_specs=ins,
        out_specs=pl.BlockSpec((nb, dmlp), lambda i: (i, 0)),
        out_shape=jax.ShapeDtypeStruct((r, dmlp), jnp.float32),
    )(*args)


def _kron_eye(w, k):
    return jnp.kron(jnp.eye(k, dtype=jnp.float32), w)


def _fold_bn(p):
    s = p['gamma'] * lax.rsqrt(p['var'] + 1e-6)
    return p['W'] * s[None, :], (p['b'] - p['mean']) * s + p['beta']


def _split_att(pfc, pmlp):
    w, b = pfc['W'], pfc['b']
    d = w.shape[0]
    dh = d // 2
    wm, bm = _fold_bn(pmlp)
    return dict(
        a1=w[:dh, :dh], b1=w[dh:, :dh], a2=w[:dh, dh:], b2=w[dh:, dh:],
        bf1=b[:dh], bf2=b[dh:], wm1=wm[:dh], wm2=wm[dh:], bm=bm,
    )


def kernel(features, xyz_0, xyz_1, xyz_2, xyz_3,
           neigh_idx_0, neigh_idx_1, neigh_idx_2, neigh_idx_3,
           sub_idx_0, sub_idx_1, sub_idx_2, sub_idx_3,
           interp_idx_0, interp_idx_1, interp_idx_2, interp_idx_3, params):
    P = params
    B = features.shape[0]
    xyzs = [xyz_0, xyz_1, xyz_2, xyz_3]
    neighs = [neigh_idx_0, neigh_idx_1, neigh_idx_2, neigh_idx_3]
    subs = [sub_idx_0, sub_idx_1, sub_idx_2, sub_idx_3]
    interps = [interp_idx_0, interp_idx_1, interp_idx_2, interp_idx_3]

    def flat_idx(idx, n_src):
        offs = (jnp.arange(B, dtype=idx.dtype) * n_src)[:, None, None]
        return (idx + offs).reshape(-1)

    w0 = P['fc0']['W']
    bn = P['bn0']
    s0 = bn['gamma'] * lax.rsqrt(bn['var'] + 1e-6)
    w0f = w0 * s0[None, :]
    b0f = (P['fc0']['b'] - bn['mean']) * s0 + bn['beta']
    f = _dense_act(features.reshape(B * _NS[0], 6), w0f, b0f, True, 0.3)

    enc = []
    for i in range(4):
        nm = 'Encoder_layer_%d' % i
        n = _NS[i]
        n_next = _NS[i + 1]
        nidx = flat_idx(neighs[i], n)
        xyz2d = xyzs[i].reshape(B * n, 3)

        w, b = _fold_bn(P[nm + 'mlp1'])
        dh = w.shape[1]
        d16 = -(-(dh + 3) // 8) * 8
        tbl1 = _mlp1_pack(f, xyz2d, w, b, d16)
        g1 = _gather_rows(tbl1, nidx)

        ap1 = _split_att(P[nm + 'LFAatt_pooling_1fc'], P[nm + 'LFAatt_pooling_1mlp'])
        ap2 = _split_att(P[nm + 'LFAatt_pooling_2fc'], P[nm + 'LFAatt_pooling_2mlp'])
        w1f, b1f = _fold_bn(P[nm + 'LFAmlp1'])
        w2f, b2f = _fold_bn(P[nm + 'LFAmlp2'])
        K = _KNN
        if i <= 1:
            rn = B * n
            g1p = g1.reshape(rn, K * d16)
            e_xyz = jnp.zeros((d16, 3), jnp.float32).at[dh:dh + 3, :].set(jnp.eye(3))
            sel_xyz = _kron_eye(e_xyz, K)
            tile3 = jnp.kron(jnp.ones((1, K), jnp.float32), jnp.eye(3, dtype=jnp.float32))
            sum3 = jnp.kron(jnp.eye(K, dtype=jnp.float32), jnp.ones((3, 1), jnp.float32))
            fx1p = _relpos_packed(xyz2d, g1p, sel_xyz, tile3, sum3,
                                  _kron_eye(w1f[0:1], K), _kron_eye(w1f[1:4], K),
                                  _kron_eye(w1f[4:7], K), _kron_eye(w1f[7:10], K),
                                  jnp.tile(b1f, K))
            ext = jnp.zeros((d16, dh), jnp.float32).at[:dh, :].set(jnp.eye(dh))
            sel_fn = _kron_eye(ext, K)
            ssum = jnp.kron(jnp.ones((K, 1), jnp.float32), jnp.eye(dh, dtype=jnp.float32))
            pad_a = lambda a: _kron_eye(jnp.concatenate(
                [a, jnp.zeros((d16 - dh, a.shape[1]), jnp.float32)], axis=0), K)
            fagg = _att_packed(g1p, fx1p,
                               pad_a(ap1['a1']), _kron_eye(ap1['b1'], K),
                               pad_a(ap1['a2']), _kron_eye(ap1['b2'], K),
                               sel_fn, ssum, jnp.tile(ap1['bf1'], K),
                               jnp.tile(ap1['bf2'], K),
                               ap1['wm1'], ap1['wm2'], ap1['bm'])
            fx2p = _dense_act(fx1p, _kron_eye(w2f, K), jnp.tile(b2f, K), True, 0.2)
            fn2p = _gather_rows(fagg, nidx).reshape(rn, K * dh)
            flfa = _att_packed(fn2p, fx2p,
                               _kron_eye(ap2['a1'], K), _kron_eye(ap2['b1'], K),
                               _kron_eye(ap2['a2'], K), _kron_eye(ap2['b2'], K),
                               None, ssum, jnp.tile(ap2['bf1'], K),
                               jnp.tile(ap2['bf2'], K),
                               ap2['wm1'], ap2['wm2'], ap2['bm'])
        else:
            fx1 = _relpos_mlp(xyz2d, g1, dh, w1f, b1f)
            fagg = _att_pool(g1, fx1, ap1['a1'], ap1['b1'], ap1['a2'], ap1['b2'],
                             ap1['bf1'], ap1['bf2'], ap1['wm1'], ap1['wm2'], ap1['bm'])
            fx2 = _dense_act(fx1, w2f, b2f, True, 0.2)
            fn2 = _gather_rows(fagg, nidx)
            flfa = _att_pool(fn2, fx2, ap2['a1'], ap2['b1'], ap2['a2'], ap2['b2'],
                             ap2['bf1'], ap2['bf2'], ap2['wm1'], ap2['wm2'], ap2['bm'])

        w2, b2 = _fold_bn(P[nm + 'mlp2'])
        ws, bs = _fold_bn(P[nm + 'shortcut'])
        fe = _dense2_act(flfa, f, w2, ws, b2 + bs, True, 0.2)


        sidx = flat_idx(subs[i], n)
        fp = _gather_rows(fe, sidx)
        f = _maxpool_k(fp, B * n_next)

        if i == 0:
            enc.append(fe)
        enc.append(f)

    w, b = _fold_bn(P['decoder_0'])
    f = _dense_act(f, w, b, True, 0.2)

    for j in range(4):
        lev = 3 - j
        n_dst = _NS[lev]
        n_src = _NS[lev + 1]
        iidx = flat_idx(interps[lev][:, :, None, 0], n_src)
        fi = _gather_rows(f, iidx)
        w, b = _fold_bn(P['Decoder_layer_%d' % j])
        skip = enc[-j - 2]
        dskip = skip.shape[1]
        f = _dense2_act(skip, fi, w[:dskip], w[dskip:], b, True, 0.2)

    w, b = _fold_bn(P['fc1'])
    f = _dense_act(f, w, b, True, 0.2)
    w, b = _fold_bn(P['fc2'])
    f = _dense_act(f, w, b, True, 0.2)
    f = _dense_act(f, P['fc']['W'], P['fc']['b'], False, 0.0)
    return f.reshape(B, _NS[0], 13)
